# baseline (speedup 1.0000x reference)
import functools

import jax
import jax.numpy as jnp
from jax import lax
from jax.experimental import pallas as pl
from jax.experimental.pallas import tpu as pltpu
from jax.experimental.pallas import tpu_sc as plsc

ITERS = 3
N_NODES = 10000
N_EDGES = 320000
D_IN = 128
D = 64
D2 = 128

NC = 2
NS = 16
NW = NC * NS
CHUNK = 128
N_CHUNKS = 80
CPP = 40
N_PHASES = N_CHUNKS // CPP
EDGES_PER_TILE = N_CHUNKS * CHUNK
E_PAD = NW * EDGES_PER_TILE
NP = 10240
RPS = NP // NS

_mesh = plsc.VectorSubcoreMesh(core_axis_name="c", subcore_axis_name="s")


@jax.jit
def _sc_aggregate(h2, src3d, dst3d):
    @functools.partial(
        pl.kernel,
        out_type=jax.ShapeDtypeStruct((NC, NP, D2), jnp.float32),
        mesh=_mesh,
        scratch_types=[
            pltpu.VMEM_SHARED((NP, D2), jnp.float32),
            pltpu.VMEM((CPP, CHUNK), jnp.int32),
            pltpu.VMEM((CPP, CHUNK), jnp.int32),
            pltpu.VMEM((CHUNK, D2), jnp.float32),
            pltpu.VMEM((CHUNK, D2), jnp.float32),
            pltpu.VMEM((16, D2), jnp.float32),
            pltpu.SemaphoreType.DMA,
            pltpu.SemaphoreType.DMA,
            pltpu.SemaphoreType.DMA,
            pltpu.SemaphoreType.DMA,
        ],
    )
    def k(h_hbm, src_hbm, dst_hbm, out_hbm,
          agg_sh, src_v, dst_v, rowsA, rowsB, zbuf, semA, semB, semSA, semSB):
        cid = lax.axis_index("c")
        sid = lax.axis_index("s")
        wid = cid * NS + sid

        def g_start(c, buf, sem):
            pltpu.make_async_copy(h_hbm.at[src_v.at[c]], buf, sem).start()

        def g_wait(c, buf, sem):
            pltpu.make_async_copy(h_hbm.at[src_v.at[c]], buf, sem).wait()

        def scat_start(buf, c, sem):
            pltpu.async_copy(buf, agg_sh.at[dst_v.at[c]], sem, add=True)

        def scat_wait(buf, c, sem):
            pltpu.make_async_copy(buf, agg_sh.at[dst_v.at[c]], sem).wait()

        pltpu.sync_copy(src_hbm.at[wid, pl.ds(0, CPP)], src_v)
        pltpu.sync_copy(dst_hbm.at[wid, pl.ds(0, CPP)], dst_v)
        g_start(0, rowsA, semA)
        g_start(1, rowsB, semB)

        @pl.loop(0, 16)
        def _(r):
            @pl.loop(0, D2, step=16)
            def _(c2):
                zbuf[r, pl.ds(c2, 16)] = jnp.zeros((16,), jnp.float32)

        @pl.loop(0, RPS // 16)
        def _(z):
            pltpu.sync_copy(zbuf, agg_sh.at[pl.ds(sid * RPS + z * 16, 16)])

        plsc.subcore_barrier()

        for p in range(N_PHASES):
            @pl.loop(0, CPP, step=2)
            def _(c):
                g_wait(c, rowsA, semA)
                scat_start(rowsA, c, semSA)
                g_wait(c + 1, rowsB, semB)
                scat_start(rowsB, c + 1, semSB)
                scat_wait(rowsA, c, semSA)

                @pl.when(c + 2 < CPP)
                def _():
                    g_start(c + 2, rowsA, semA)

                scat_wait(rowsB, c + 1, semSB)

                @pl.when(c + 3 < CPP)
                def _():
                    g_start(c + 3, rowsB, semB)

            if p + 1 < N_PHASES:
                pltpu.sync_copy(src_hbm.at[wid, pl.ds((p + 1) * CPP, CPP)], src_v)
                pltpu.sync_copy(dst_hbm.at[wid, pl.ds((p + 1) * CPP, CPP)], dst_v)
                g_start(0, rowsA, semA)
                g_start(1, rowsB, semB)

        plsc.subcore_barrier()

        pltpu.sync_copy(agg_sh.at[pl.ds(sid * RPS, RPS)],
                        out_hbm.at[cid, pl.ds(sid * RPS, RPS)])

    return k(h2, src3d, dst3d)


def _encode_body(x_ref, w_ref, b_ref, o_ref):
    h = jnp.maximum(
        jnp.dot(x_ref[...], w_ref[...], preferred_element_type=jnp.float32,
                precision=lax.Precision.HIGHEST) + b_ref[...], 0.0)
    o_ref[...] = jnp.pad(h, ((0, 0), (0, D2 - D)))


@jax.jit
def _tc_encode(x, W_in, b_in):
    BM = 1024
    return pl.pallas_call(
        _encode_body,
        grid=(NP // BM,),
        in_specs=[
            pl.BlockSpec((BM, D_IN), lambda i: (i, 0)),
            pl.BlockSpec((D_IN, D), lambda i: (0, 0)),
            pl.BlockSpec((1, D), lambda i: (0, 0)),
        ],
        out_specs=pl.BlockSpec((BM, D2), lambda i: (i, 0)),
        out_shape=jax.ShapeDtypeStruct((NP, D2), jnp.float32),
    )(x, W_in, b_in.reshape(1, D))


def _update_last_body(h_ref, p0_ref, p1_ref, ws_ref, wm_ref, b_ref, o_ref):
    agg = p0_ref[:, :D] + p1_ref[:, :D]
    o_ref[...] = jnp.maximum(
        jnp.dot(h_ref[:, :D], ws_ref[...], preferred_element_type=jnp.float32,
                precision=lax.Precision.HIGHEST)
        + jnp.dot(agg, wm_ref[...], preferred_element_type=jnp.float32,
                  precision=lax.Precision.HIGHEST)
        + b_ref[...], 0.0)


@jax.jit
def _tc_update_last(h2, partials, W_self, W_msg, b_upd):
    BM = 1024
    return pl.pallas_call(
        _update_last_body,
        grid=(NP // BM,),
        in_specs=[
            pl.BlockSpec((BM, D2), lambda i: (i, 0)),
            pl.BlockSpec((BM, D2), lambda i: (i, 0)),
            pl.BlockSpec((BM, D2), lambda i: (i, 0)),
            pl.BlockSpec((D, D), lambda i: (0, 0)),
            pl.BlockSpec((D, D), lambda i: (0, 0)),
            pl.BlockSpec((1, D), lambda i: (0, 0)),
        ],
        out_specs=pl.BlockSpec((BM, D), lambda i: (i, 0)),
        out_shape=jax.ShapeDtypeStruct((N_NODES, D), jnp.float32),
    )(h2, partials[0], partials[1], W_self, W_msg, b_upd.reshape(1, D))


def _update_body(h_ref, p0_ref, p1_ref, ws_ref, wm_ref, b_ref, o_ref):
    agg = p0_ref[:, :D] + p1_ref[:, :D]
    h = jnp.maximum(
        jnp.dot(h_ref[:, :D], ws_ref[...], preferred_element_type=jnp.float32,
                precision=lax.Precision.HIGHEST)
        + jnp.dot(agg, wm_ref[...], preferred_element_type=jnp.float32,
                  precision=lax.Precision.HIGHEST)
        + b_ref[...], 0.0)
    o_ref[...] = jnp.pad(h, ((0, 0), (0, D2 - D)))


@jax.jit
def _tc_update(h2, partials, W_self, W_msg, b_upd):
    BM = 1024
    return pl.pallas_call(
        _update_body,
        grid=(NP // BM,),
        in_specs=[
            pl.BlockSpec((BM, D2), lambda i: (i, 0)),
            pl.BlockSpec((BM, D2), lambda i: (i, 0)),
            pl.BlockSpec((BM, D2), lambda i: (i, 0)),
            pl.BlockSpec((D, D), lambda i: (0, 0)),
            pl.BlockSpec((D, D), lambda i: (0, 0)),
            pl.BlockSpec((1, D), lambda i: (0, 0)),
        ],
        out_specs=pl.BlockSpec((BM, D2), lambda i: (i, 0)),
        out_shape=jax.ShapeDtypeStruct((NP, D2), jnp.float32),
    )(h2, partials[0], partials[1], W_self, W_msg, b_upd.reshape(1, D))


def kernel(x, edge_index, W_in, b_in, W_msg, W_self, b_upd):
    n_dummy = E_PAD - N_EDGES
    pad_idx = N_NODES + (jnp.arange(n_dummy, dtype=jnp.int32) % (NP - N_NODES))
    src3d = jnp.concatenate([edge_index[0], pad_idx]).reshape(NW, N_CHUNKS, CHUNK)
    dst3d = jnp.concatenate([edge_index[1], pad_idx]).reshape(NW, N_CHUNKS, CHUNK)
    h2 = _tc_encode(x, W_in, b_in)
    for _ in range(ITERS - 1):
        partials = _sc_aggregate(h2, src3d, dst3d)
        h2 = _tc_update(h2, partials, W_self, W_msg, b_upd)
    partials = _sc_aggregate(h2, src3d, dst3d)
    return _tc_update_last(h2, partials, W_self, W_msg, b_upd)


# speedup vs baseline: 1.2455x; 1.2455x over previous
import functools

import jax
import jax.numpy as jnp
from jax import lax
from jax.experimental import pallas as pl
from jax.experimental.pallas import tpu as pltpu
from jax.experimental.pallas import tpu_sc as plsc

ITERS = 3
N_NODES = 10000
N_EDGES = 320000
D_IN = 128
D = 64
D2 = 128

NC = 2
NS = 16
NW = NC * NS
CHUNK = 128
N_CHUNKS = 80
CPP = 40
N_PHASES = N_CHUNKS // CPP
EDGES_PER_TILE = N_CHUNKS * CHUNK
E_PAD = NW * EDGES_PER_TILE
NP = 10240
RPS = NP // NS

_mesh = plsc.VectorSubcoreMesh(core_axis_name="c", subcore_axis_name="s")


@jax.jit
def _sc_aggregate(h2, src3d, dst3d):
    @functools.partial(
        pl.kernel,
        out_type=jax.ShapeDtypeStruct((NC, NP, D2), jnp.float32),
        mesh=_mesh,
        scratch_types=[
            pltpu.VMEM_SHARED((NP, D2), jnp.float32),
            pltpu.VMEM((CPP, CHUNK), jnp.int32),
            pltpu.VMEM((CPP, CHUNK), jnp.int32),
            pltpu.VMEM((CHUNK, D2), jnp.float32),
            pltpu.VMEM((CHUNK, D2), jnp.float32),
            pltpu.VMEM((16, D2), jnp.float32),
            pltpu.SemaphoreType.DMA,
            pltpu.SemaphoreType.DMA,
        ],
    )
    def k(h_hbm, src_hbm, dst_hbm, out_hbm,
          agg_sh, src_v, dst_v, rowsA, rowsB, zbuf, semA, semB):
        cid = lax.axis_index("c")
        sid = lax.axis_index("s")
        wid = cid * NS + sid

        def g_start(c, buf, sem):
            pltpu.make_async_copy(h_hbm.at[src_v.at[c]], buf, sem).start()

        def g_wait(c, buf, sem):
            pltpu.make_async_copy(h_hbm.at[src_v.at[c]], buf, sem).wait()

        def scat(buf, c):
            pltpu.sync_copy(buf, agg_sh.at[dst_v.at[c]], add=True)

        pltpu.sync_copy(src_hbm.at[wid, pl.ds(0, CPP)], src_v)
        pltpu.sync_copy(dst_hbm.at[wid, pl.ds(0, CPP)], dst_v)
        g_start(0, rowsA, semA)
        g_start(1, rowsB, semB)

        @pl.loop(0, 16)
        def _(r):
            @pl.loop(0, D2, step=16)
            def _(c2):
                zbuf[r, pl.ds(c2, 16)] = jnp.zeros((16,), jnp.float32)

        @pl.loop(0, RPS // 16)
        def _(z):
            pltpu.sync_copy(zbuf, agg_sh.at[pl.ds(sid * RPS + z * 16, 16)])

        plsc.subcore_barrier()

        for p in range(N_PHASES):
            @pl.loop(0, CPP, step=2)
            def _(c):
                g_wait(c, rowsA, semA)
                scat(rowsA, c)

                @pl.when(c + 2 < CPP)
                def _():
                    g_start(c + 2, rowsA, semA)

                g_wait(c + 1, rowsB, semB)
                scat(rowsB, c + 1)

                @pl.when(c + 3 < CPP)
                def _():
                    g_start(c + 3, rowsB, semB)

            if p + 1 < N_PHASES:
                pltpu.sync_copy(src_hbm.at[wid, pl.ds((p + 1) * CPP, CPP)], src_v)
                pltpu.sync_copy(dst_hbm.at[wid, pl.ds((p + 1) * CPP, CPP)], dst_v)
                g_start(0, rowsA, semA)
                g_start(1, rowsB, semB)

        plsc.subcore_barrier()

        pltpu.sync_copy(agg_sh.at[pl.ds(sid * RPS, RPS)],
                        out_hbm.at[cid, pl.ds(sid * RPS, RPS)])

    return k(h2, src3d, dst3d)


def _encode_body(x_ref, w_ref, b_ref, o_ref):
    h = jnp.maximum(
        jnp.dot(x_ref[...], w_ref[...], preferred_element_type=jnp.float32,
                precision=lax.Precision.HIGHEST) + b_ref[...], 0.0)
    o_ref[...] = jnp.pad(h, ((0, 0), (0, D2 - D)))


@jax.jit
def _tc_encode(x, W_in, b_in):
    BM = 1024
    return pl.pallas_call(
        _encode_body,
        grid=(NP // BM,),
        in_specs=[
            pl.BlockSpec((BM, D_IN), lambda i: (i, 0)),
            pl.BlockSpec((D_IN, D), lambda i: (0, 0)),
            pl.BlockSpec((1, D), lambda i: (0, 0)),
        ],
        out_specs=pl.BlockSpec((BM, D2), lambda i: (i, 0)),
        out_shape=jax.ShapeDtypeStruct((NP, D2), jnp.float32),
    )(x, W_in, b_in.reshape(1, D))


def _update_last_body(h_ref, p0_ref, p1_ref, ws_ref, wm_ref, b_ref, o_ref):
    agg = p0_ref[:, :D] + p1_ref[:, :D]
    o_ref[...] = jnp.maximum(
        jnp.dot(h_ref[:, :D], ws_ref[...], preferred_element_type=jnp.float32,
                precision=lax.Precision.HIGHEST)
        + jnp.dot(agg, wm_ref[...], preferred_element_type=jnp.float32,
                  precision=lax.Precision.HIGHEST)
        + b_ref[...], 0.0)


@jax.jit
def _tc_update_last(h2, partials, W_self, W_msg, b_upd):
    BM = 1024
    return pl.pallas_call(
        _update_last_body,
        grid=(NP // BM,),
        in_specs=[
            pl.BlockSpec((BM, D2), lambda i: (i, 0)),
            pl.BlockSpec((BM, D2), lambda i: (i, 0)),
            pl.BlockSpec((BM, D2), lambda i: (i, 0)),
            pl.BlockSpec((D, D), lambda i: (0, 0)),
            pl.BlockSpec((D, D), lambda i: (0, 0)),
            pl.BlockSpec((1, D), lambda i: (0, 0)),
        ],
        out_specs=pl.BlockSpec((BM, D), lambda i: (i, 0)),
        out_shape=jax.ShapeDtypeStruct((N_NODES, D), jnp.float32),
    )(h2, partials[0], partials[1], W_self, W_msg, b_upd.reshape(1, D))


def _update_body(h_ref, p0_ref, p1_ref, ws_ref, wm_ref, b_ref, o_ref):
    agg = p0_ref[:, :D] + p1_ref[:, :D]
    h = jnp.maximum(
        jnp.dot(h_ref[:, :D], ws_ref[...], preferred_element_type=jnp.float32,
                precision=lax.Precision.HIGHEST)
        + jnp.dot(agg, wm_ref[...], preferred_element_type=jnp.float32,
                  precision=lax.Precision.HIGHEST)
        + b_ref[...], 0.0)
    o_ref[...] = jnp.pad(h, ((0, 0), (0, D2 - D)))


@jax.jit
def _tc_update(h2, partials, W_self, W_msg, b_upd):
    BM = 1024
    return pl.pallas_call(
        _update_body,
        grid=(NP // BM,),
        in_specs=[
            pl.BlockSpec((BM, D2), lambda i: (i, 0)),
            pl.BlockSpec((BM, D2), lambda i: (i, 0)),
            pl.BlockSpec((BM, D2), lambda i: (i, 0)),
            pl.BlockSpec((D, D), lambda i: (0, 0)),
            pl.BlockSpec((D, D), lambda i: (0, 0)),
            pl.BlockSpec((1, D), lambda i: (0, 0)),
        ],
        out_specs=pl.BlockSpec((BM, D2), lambda i: (i, 0)),
        out_shape=jax.ShapeDtypeStruct((NP, D2), jnp.float32),
    )(h2, partials[0], partials[1], W_self, W_msg, b_upd.reshape(1, D))


def kernel(x, edge_index, W_in, b_in, W_msg, W_self, b_upd):
    n_dummy = E_PAD - N_EDGES
    pad_idx = N_NODES + (jnp.arange(n_dummy, dtype=jnp.int32) % (NP - N_NODES))
    src3d = jnp.concatenate([edge_index[0], pad_idx]).reshape(NW, N_CHUNKS, CHUNK)
    dst3d = jnp.concatenate([edge_index[1], pad_idx]).reshape(NW, N_CHUNKS, CHUNK)
    h2 = _tc_encode(x, W_in, b_in)
    for _ in range(ITERS - 1):
        partials = _sc_aggregate(h2, src3d, dst3d)
        h2 = _tc_update(h2, partials, W_self, W_msg, b_upd)
    partials = _sc_aggregate(h2, src3d, dst3d)
    return _tc_update_last(h2, partials, W_self, W_msg, b_upd)


# speedup vs baseline: 1.3135x; 1.0546x over previous
import functools

import jax
import jax.numpy as jnp
from jax import lax
from jax.experimental import pallas as pl
from jax.experimental.pallas import tpu as pltpu
from jax.experimental.pallas import tpu_sc as plsc

ITERS = 3
N_NODES = 10000
N_EDGES = 320000
D_IN = 128
D = 64
D2 = 128

NC = 2
NS = 16
NW = NC * NS
CHUNK = 128
N_CHUNKS = 80
CPP = 40
N_PHASES = N_CHUNKS // CPP
EDGES_PER_TILE = N_CHUNKS * CHUNK
E_PAD = NW * EDGES_PER_TILE
NP = 10240
RPS = NP // NS

_mesh = plsc.VectorSubcoreMesh(core_axis_name="c", subcore_axis_name="s")


@jax.jit
def _sc_aggregate(h2, src3d, dst3d):
    @functools.partial(
        pl.kernel,
        out_type=jax.ShapeDtypeStruct((NC, NP, D2), jnp.float32),
        mesh=_mesh,
        scratch_types=[
            pltpu.VMEM_SHARED((NP, D2), jnp.float32),
            pltpu.VMEM((CPP, CHUNK), jnp.int32),
            pltpu.VMEM((CPP, CHUNK), jnp.int32),
            pltpu.VMEM((CHUNK, D2), jnp.float32),
            pltpu.VMEM((CHUNK, D2), jnp.float32),
            pltpu.VMEM((16, D2), jnp.float32),
            pltpu.SemaphoreType.DMA,
            pltpu.SemaphoreType.DMA,
        ],
    )
    def k(h_hbm, src_hbm, dst_hbm, out_hbm,
          agg_sh, src_v, dst_v, rowsA, rowsB, zbuf, semA, semB):
        cid = lax.axis_index("c")
        sid = lax.axis_index("s")
        wid = cid * NS + sid

        def g_start(c, buf, sem):
            pltpu.make_async_copy(h_hbm.at[src_v.at[c]], buf, sem).start()

        def g_wait(c, buf, sem):
            pltpu.make_async_copy(h_hbm.at[src_v.at[c]], buf, sem).wait()

        def scat(buf, c):
            pltpu.sync_copy(buf, agg_sh.at[dst_v.at[c]], add=True)

        pltpu.sync_copy(src_hbm.at[wid, pl.ds(0, CPP)], src_v)
        pltpu.sync_copy(dst_hbm.at[wid, pl.ds(0, CPP)], dst_v)
        g_start(0, rowsA, semA)
        g_start(1, rowsB, semB)

        @pl.loop(0, 16)
        def _(r):
            @pl.loop(0, D2, step=16)
            def _(c2):
                zbuf[r, pl.ds(c2, 16)] = jnp.zeros((16,), jnp.float32)

        @pl.loop(0, RPS // 16)
        def _(z):
            pltpu.sync_copy(zbuf, agg_sh.at[pl.ds(sid * RPS + z * 16, 16)])

        plsc.subcore_barrier()

        for p in range(N_PHASES):
            @pl.loop(0, CPP, step=2)
            def _(c):
                g_wait(c, rowsA, semA)
                scat(rowsA, c)

                @pl.when(c + 2 < CPP)
                def _():
                    g_start(c + 2, rowsA, semA)

                g_wait(c + 1, rowsB, semB)
                scat(rowsB, c + 1)

                @pl.when(c + 3 < CPP)
                def _():
                    g_start(c + 3, rowsB, semB)

            if p + 1 < N_PHASES:
                pltpu.sync_copy(src_hbm.at[wid, pl.ds((p + 1) * CPP, CPP)], src_v)
                pltpu.sync_copy(dst_hbm.at[wid, pl.ds((p + 1) * CPP, CPP)], dst_v)
                g_start(0, rowsA, semA)
                g_start(1, rowsB, semB)

        plsc.subcore_barrier()

        pltpu.sync_copy(agg_sh.at[pl.ds(sid * RPS, RPS)],
                        out_hbm.at[cid, pl.ds(sid * RPS, RPS)])

    return k(h2, src3d, dst3d)


def _encode_body(x_ref, w_ref, b_ref, o_ref):
    h = jnp.maximum(
        jnp.dot(x_ref[...], w_ref[...], preferred_element_type=jnp.float32,
                precision=lax.Precision.HIGHEST) + b_ref[...], 0.0)
    o_ref[...] = jnp.pad(h, ((0, 0), (0, D2 - D)))


@jax.jit
def _tc_encode(x, W_in, b_in):
    BM = 1024
    return pl.pallas_call(
        _encode_body,
        grid=(NP // BM,),
        in_specs=[
            pl.BlockSpec((BM, D_IN), lambda i: (i, 0)),
            pl.BlockSpec((D_IN, D), lambda i: (0, 0)),
            pl.BlockSpec((1, D), lambda i: (0, 0)),
        ],
        out_specs=pl.BlockSpec((BM, D2), lambda i: (i, 0)),
        out_shape=jax.ShapeDtypeStruct((NP, D2), jnp.float32),
    )(x, W_in, b_in.reshape(1, D))


def _update_last_body(h_ref, p0_ref, p1_ref, ws_ref, wm_ref, b_ref, o_ref):
    agg = p0_ref[0, :, :D] + p1_ref[0, :, :D]
    o_ref[...] = jnp.maximum(
        jnp.dot(h_ref[:, :D], ws_ref[...], preferred_element_type=jnp.float32,
                precision=lax.Precision.HIGHEST)
        + jnp.dot(agg, wm_ref[...], preferred_element_type=jnp.float32,
                  precision=lax.Precision.HIGHEST)
        + b_ref[...], 0.0)


@jax.jit
def _tc_update_last(h2, partials, W_self, W_msg, b_upd):
    BM = 1024
    return pl.pallas_call(
        _update_last_body,
        grid=(NP // BM,),
        in_specs=[
            pl.BlockSpec((BM, D2), lambda i: (i, 0)),
            pl.BlockSpec((1, BM, D2), lambda i: (0, i, 0)),
            pl.BlockSpec((1, BM, D2), lambda i: (1, i, 0)),
            pl.BlockSpec((D, D), lambda i: (0, 0)),
            pl.BlockSpec((D, D), lambda i: (0, 0)),
            pl.BlockSpec((1, D), lambda i: (0, 0)),
        ],
        out_specs=pl.BlockSpec((BM, D), lambda i: (i, 0)),
        out_shape=jax.ShapeDtypeStruct((N_NODES, D), jnp.float32),
    )(h2, partials, partials, W_self, W_msg, b_upd.reshape(1, D))


def _update_body(h_ref, p0_ref, p1_ref, ws_ref, wm_ref, b_ref, o_ref):
    agg = p0_ref[0, :, :D] + p1_ref[0, :, :D]
    h = jnp.maximum(
        jnp.dot(h_ref[:, :D], ws_ref[...], preferred_element_type=jnp.float32,
                precision=lax.Precision.HIGHEST)
        + jnp.dot(agg, wm_ref[...], preferred_element_type=jnp.float32,
                  precision=lax.Precision.HIGHEST)
        + b_ref[...], 0.0)
    o_ref[...] = jnp.pad(h, ((0, 0), (0, D2 - D)))


@jax.jit
def _tc_update(h2, partials, W_self, W_msg, b_upd):
    BM = 1024
    return pl.pallas_call(
        _update_body,
        grid=(NP // BM,),
        in_specs=[
            pl.BlockSpec((BM, D2), lambda i: (i, 0)),
            pl.BlockSpec((1, BM, D2), lambda i: (0, i, 0)),
            pl.BlockSpec((1, BM, D2), lambda i: (1, i, 0)),
            pl.BlockSpec((D, D), lambda i: (0, 0)),
            pl.BlockSpec((D, D), lambda i: (0, 0)),
            pl.BlockSpec((1, D), lambda i: (0, 0)),
        ],
        out_specs=pl.BlockSpec((BM, D2), lambda i: (i, 0)),
        out_shape=jax.ShapeDtypeStruct((NP, D2), jnp.float32),
    )(h2, partials, partials, W_self, W_msg, b_upd.reshape(1, D))


def kernel(x, edge_index, W_in, b_in, W_msg, W_self, b_upd):
    n_dummy = E_PAD - N_EDGES
    pad_idx = N_NODES + (jnp.arange(n_dummy, dtype=jnp.int32) % (NP - N_NODES))
    src3d = jnp.concatenate([edge_index[0], pad_idx]).reshape(NW, N_CHUNKS, CHUNK)
    dst3d = jnp.concatenate([edge_index[1], pad_idx]).reshape(NW, N_CHUNKS, CHUNK)
    h2 = _tc_encode(x, W_in, b_in)
    for _ in range(ITERS - 1):
        partials = _sc_aggregate(h2, src3d, dst3d)
        h2 = _tc_update(h2, partials, W_self, W_msg, b_upd)
    partials = _sc_aggregate(h2, src3d, dst3d)
    return _tc_update_last(h2, partials, W_self, W_msg, b_upd)
